# Initial kernel scaffold; baseline (speedup 1.0000x reference)
import jax
import jax.numpy as jnp
from jax.experimental import pallas as pl


def kernel(inputs, hidden_state, supports, W_gate0, b_gate0, W_cand0, b_cand0, W_gate1, b_gate1, W_cand1, b_cand1, W_pred, b_pred):
    raise NotImplementedError("write your pallas kernel here")


# speedup vs baseline: 1.4823x; 1.4823x over previous
import functools

import jax
import jax.numpy as jnp
from jax.experimental import pallas as pl
from jax.experimental.pallas import tpu as pltpu

UNITS = 64
NMAT = 5
BM = 512


def _gconv_body(S_ref, xin_ref, h_ref, *rest, is_cand, with_pred, nb):
    if is_cand:
        gate_ref, *rest = rest
    W_ref, b_ref, *rest = rest
    if with_pred:
        wp_ref, bp_ref, out_ref, pred_ref, *rest = rest
    else:
        out_ref, *rest = rest
    x0b, x1a, x1b, x2a, x2b = rest

    p = pl.program_id(0)
    i = pl.program_id(1)
    blk = pl.ds(i * BM, BM)
    f32 = jnp.float32

    @pl.when((p == 0) & (i == 0))
    def _init():
        if is_cand:
            st = gate_ref[:, :UNITS] * h_ref[...]
        else:
            st = h_ref[...]
        x0b[...] = jnp.concatenate([xin_ref[...], st], axis=1).astype(jnp.bfloat16)

    Sa = S_ref[0]
    Sb = S_ref[1]

    @pl.when(p == 0)
    def _pass1():
        x1a[blk, :] = jnp.dot(Sa, x0b[...], preferred_element_type=f32).astype(jnp.bfloat16)
        x1b[blk, :] = jnp.dot(Sb, x0b[...], preferred_element_type=f32).astype(jnp.bfloat16)

    @pl.when(p == 1)
    def _pass2():
        x0f = x0b[blk, :].astype(f32)
        x2a[blk, :] = (2.0 * jnp.dot(Sa, x1a[...], preferred_element_type=f32) - x0f).astype(jnp.bfloat16)
        x2b[blk, :] = (2.0 * jnp.dot(Sb, x1b[...], preferred_element_type=f32) - x0f).astype(jnp.bfloat16)

    @pl.when((p == 1) & (i == nb - 1))
    def _finish():
        Wb = W_ref[...].astype(jnp.bfloat16)
        acc = b_ref[...] + jnp.dot(x0b[...], Wb[0], preferred_element_type=f32)
        acc = acc + jnp.dot(x1a[...], Wb[1], preferred_element_type=f32)
        acc = acc + jnp.dot(x2a[...], Wb[2], preferred_element_type=f32)
        acc = acc + jnp.dot(x1b[...], Wb[3], preferred_element_type=f32)
        acc = acc + jnp.dot(x2b[...], Wb[4], preferred_element_type=f32)
        if not is_cand:
            out_ref[...] = jax.nn.sigmoid(acc)
        else:
            c = jnp.tanh(acc)
            u = gate_ref[:, UNITS:]
            hn = u * h_ref[...] + (1.0 - u) * c
            out_ref[...] = hn
            if with_pred:
                pred_ref[...] = jnp.dot(hn, wp_ref[...], preferred_element_type=f32) + bp_ref[...]


def _gconv(S2, xin, h, gate, Wr, b, wp=None, bp=None):
    n = S2.shape[1]
    nb = n // BM
    is_cand = gate is not None
    with_pred = wp is not None
    din = xin.shape[1]
    d = Wr.shape[1]
    out = Wr.shape[2]

    const = lambda *shape: pl.BlockSpec(shape, lambda p, i: (0,) * len(shape))
    in_specs = [
        pl.BlockSpec((2, BM, n), lambda p, i: (0, i, 0)),
        const(n, din),
        const(n, UNITS),
    ]
    operands = [S2, xin, h]
    if is_cand:
        in_specs.append(const(n, 2 * UNITS))
        operands.append(gate)
    in_specs += [const(NMAT, d, out), const(1, out)]
    operands += [Wr, b.reshape(1, out)]
    out_shape = jax.ShapeDtypeStruct((n, out), jnp.float32)
    out_specs = const(n, out)
    if with_pred:
        in_specs += [const(UNITS, 1), const(1, 1)]
        operands += [wp, bp.reshape(1, 1)]
        out_shape = [out_shape, jax.ShapeDtypeStruct((n, 1), jnp.float32)]
        out_specs = [out_specs, const(n, 1)]

    body = functools.partial(_gconv_body, is_cand=is_cand, with_pred=with_pred, nb=nb)
    return pl.pallas_call(
        body,
        grid=(2, nb),
        in_specs=in_specs,
        out_specs=out_specs,
        out_shape=out_shape,
        scratch_shapes=[pltpu.VMEM((n, d), jnp.bfloat16)] * 5,
        compiler_params=pltpu.CompilerParams(
            dimension_semantics=("arbitrary", "arbitrary")),
    )(*operands)


def _split_w(W, d, out):
    return W.reshape(d, NMAT, out).transpose(1, 0, 2)


def kernel(inputs, hidden_state, supports, W_gate0, b_gate0, W_cand0, b_cand0,
           W_gate1, b_gate1, W_cand1, b_cand1, W_pred, b_pred):
    n = supports.shape[1]
    S2 = supports.astype(jnp.bfloat16)
    xin = inputs[0]
    h0 = hidden_state[0, 0]
    h1 = hidden_state[1, 0]
    d0 = xin.shape[1] + UNITS
    d1 = 2 * UNITS

    gate0 = _gconv(S2, xin, h0, None, _split_w(W_gate0, d0, 2 * UNITS), b_gate0)
    h0n = _gconv(S2, xin, h0, gate0, _split_w(W_cand0, d0, UNITS), b_cand0)
    gate1 = _gconv(S2, h0n, h1, None, _split_w(W_gate1, d1, 2 * UNITS), b_gate1)
    h1n, pred = _gconv(S2, h0n, h1, gate1, _split_w(W_cand1, d1, UNITS), b_cand1,
                       W_pred, b_pred)

    return pred[None], jnp.stack([h0n, h1n])[:, None]


# speedup vs baseline: 1.5038x; 1.0145x over previous
import functools

import jax
import jax.numpy as jnp
from jax.experimental import pallas as pl
from jax.experimental.pallas import tpu as pltpu

UNITS = 64
NMAT = 5
BM = 512


def _gconv_body(S_hbm, xin_ref, h_ref, *rest, is_cand, with_pred, nb, n):
    if is_cand:
        gate_ref, *rest = rest
    W_ref, b_ref, *rest = rest
    if with_pred:
        wp_ref, bp_ref, out_ref, pred_ref, *rest = rest
    else:
        out_ref, *rest = rest
    Sv, x0b, x1a, x1b, x2a, x2b, sems = rest

    s = pl.program_id(0)
    p = pl.program_id(1)
    i = pl.program_id(2)
    blk = pl.ds(i * BM, BM)
    f32 = jnp.float32

    def s_copy(sup, j):
        return pltpu.make_async_copy(
            S_hbm.at[sup, pl.ds(j * BM, BM), :], Sv.at[pl.ds(j * BM, BM), :],
            sems.at[j])

    @pl.when((s == 0) & (p == 0) & (i == 0))
    def _init():
        if is_cand:
            st = gate_ref[:, :UNITS] * h_ref[...]
        else:
            st = h_ref[...]
        x0b[...] = jnp.concatenate([xin_ref[...], st], axis=1).astype(jnp.bfloat16)
        for j in range(nb):
            s_copy(0, j).start()

    @pl.when(p == 0)
    def _wait():
        s_copy(s, i).wait()

    Sblk = Sv[blk, :]

    @pl.when((s == 0) & (p == 0))
    def _a1():
        x1a[blk, :] = jnp.dot(Sblk, x0b[...], preferred_element_type=f32).astype(jnp.bfloat16)

    @pl.when((s == 0) & (p == 1))
    def _a2():
        x2a[blk, :] = (2.0 * jnp.dot(Sblk, x1a[...], preferred_element_type=f32)
                       - x0b[blk, :].astype(f32)).astype(jnp.bfloat16)

    @pl.when((s == 1) & (p == 0))
    def _b1():
        x1b[blk, :] = jnp.dot(Sblk, x0b[...], preferred_element_type=f32).astype(jnp.bfloat16)

    @pl.when((s == 1) & (p == 1))
    def _b2():
        x2b[blk, :] = (2.0 * jnp.dot(Sblk, x1b[...], preferred_element_type=f32)
                       - x0b[blk, :].astype(f32)).astype(jnp.bfloat16)

    @pl.when((s == 0) & (p == 1) & (i > 0))
    def _refill():
        s_copy(1, i - 1).start()

    @pl.when((s == 1) & (p == 0) & (i == 0))
    def _refill_last():
        s_copy(1, nb - 1).start()

    @pl.when((s == 1) & (p == 1) & (i == nb - 1))
    def _finish():
        Wb = W_ref[...].astype(jnp.bfloat16)
        acc = b_ref[...] + jnp.dot(x0b[...], Wb[0], preferred_element_type=f32)
        acc = acc + jnp.dot(x1a[...], Wb[1], preferred_element_type=f32)
        acc = acc + jnp.dot(x2a[...], Wb[2], preferred_element_type=f32)
        acc = acc + jnp.dot(x1b[...], Wb[3], preferred_element_type=f32)
        acc = acc + jnp.dot(x2b[...], Wb[4], preferred_element_type=f32)
        if not is_cand:
            out_ref[...] = jax.nn.sigmoid(acc)
        else:
            c = jnp.tanh(acc)
            u = gate_ref[:, UNITS:]
            hn = u * h_ref[...] + (1.0 - u) * c
            out_ref[...] = hn
            if with_pred:
                pred_ref[...] = jnp.dot(hn, wp_ref[...], preferred_element_type=f32) + bp_ref[...]


def _gconv(S2, xin, h, gate, Wr, b, wp=None, bp=None):
    n = S2.shape[1]
    nb = n // BM
    is_cand = gate is not None
    with_pred = wp is not None
    din = xin.shape[1]
    d = Wr.shape[1]
    out = Wr.shape[2]

    const = lambda *shape: pl.BlockSpec(shape, lambda s, p, i: (0,) * len(shape))
    in_specs = [
        pl.BlockSpec(memory_space=pl.ANY),
        const(n, din),
        const(n, UNITS),
    ]
    operands = [S2, xin, h]
    if is_cand:
        in_specs.append(const(n, 2 * UNITS))
        operands.append(gate)
    in_specs += [const(NMAT, d, out), const(1, out)]
    operands += [Wr, b.reshape(1, out)]
    out_shape = jax.ShapeDtypeStruct((n, out), jnp.float32)
    out_specs = const(n, out)
    if with_pred:
        in_specs += [const(UNITS, 1), const(1, 1)]
        operands += [wp, bp.reshape(1, 1)]
        out_shape = [out_shape, jax.ShapeDtypeStruct((n, 1), jnp.float32)]
        out_specs = [out_specs, const(n, 1)]

    body = functools.partial(_gconv_body, is_cand=is_cand, with_pred=with_pred,
                             nb=nb, n=n)
    return pl.pallas_call(
        body,
        grid=(2, 2, nb),
        in_specs=in_specs,
        out_specs=out_specs,
        out_shape=out_shape,
        scratch_shapes=[
            pltpu.VMEM((n, n), jnp.bfloat16),
            pltpu.VMEM((n, d), jnp.bfloat16),
            pltpu.VMEM((n, d), jnp.bfloat16),
            pltpu.VMEM((n, d), jnp.bfloat16),
            pltpu.VMEM((n, d), jnp.bfloat16),
            pltpu.VMEM((n, d), jnp.bfloat16),
            pltpu.SemaphoreType.DMA((nb,)),
        ],
        compiler_params=pltpu.CompilerParams(
            dimension_semantics=("arbitrary", "arbitrary", "arbitrary")),
    )(*operands)


def _split_w(W, d, out):
    return W.reshape(d, NMAT, out).transpose(1, 0, 2)


def kernel(inputs, hidden_state, supports, W_gate0, b_gate0, W_cand0, b_cand0,
           W_gate1, b_gate1, W_cand1, b_cand1, W_pred, b_pred):
    n = supports.shape[1]
    S2 = supports.astype(jnp.bfloat16)
    xin = inputs[0]
    h0 = hidden_state[0, 0]
    h1 = hidden_state[1, 0]
    d0 = xin.shape[1] + UNITS
    d1 = 2 * UNITS

    gate0 = _gconv(S2, xin, h0, None, _split_w(W_gate0, d0, 2 * UNITS), b_gate0)
    h0n = _gconv(S2, xin, h0, gate0, _split_w(W_cand0, d0, UNITS), b_cand0)
    gate1 = _gconv(S2, h0n, h1, None, _split_w(W_gate1, d1, 2 * UNITS), b_gate1)
    h1n, pred = _gconv(S2, h0n, h1, gate1, _split_w(W_cand1, d1, UNITS), b_cand1,
                       W_pred, b_pred)

    return pred[None], jnp.stack([h0n, h1n])[:, None]
